# Initial kernel scaffold; baseline (speedup 1.0000x reference)
import jax
import jax.numpy as jnp
from jax.experimental import pallas as pl


def kernel(x, t, b, pool_ids, g1, be1, W1, b1, Wt, bt, g2, be2, W2, b2, Wd, bd):
    raise NotImplementedError("write your pallas kernel here")


# speedup vs baseline: 3.4500x; 3.4500x over previous
import functools

import jax
import jax.numpy as jnp
from jax import lax
from jax.experimental import pallas as pl
from jax.experimental.pallas import tpu as pltpu
from jax.experimental.pallas import tpu_sc as plsc

N = 100000
C = 128
NB = 16
NPOOL = 12500

TILE = 2000
GRID = N // TILE

NW = 32
CHUNK = 128
CPW = 25
NP_PAD = NW * CPW * CHUNK
RPAD = 12800
RSUB = RPAD // 16


def _silu(v):
    return v * jax.nn.sigmoid(v)


def _h_tile(xb, bidx, mu1, r1, g1, be1, W1, b1, tproj):
    a = (xb - mu1) * r1 * g1 + be1
    a = _silu(a)
    h1 = jnp.dot(a, W1, preferred_element_type=jnp.float32) + b1
    onehot = (bidx[:, None] == lax.broadcasted_iota(jnp.int32, (TILE, NB), 1)
              ).astype(jnp.float32)
    sc = jnp.dot(onehot, tproj[:, :C], preferred_element_type=jnp.float32)
    sh = jnp.dot(onehot, tproj[:, C:], preferred_element_type=jnp.float32)
    return (1.0 + sc) * h1 + sh


def _p1_body(x_ref, t_ref, Wt_ref, bt_ref, sums_ref, tproj_ref, a1, a2):
    i = pl.program_id(0)

    @pl.when(i == 0)
    def _init():
        a1[...] = jnp.zeros_like(a1)
        a2[...] = jnp.zeros_like(a2)
        tt = t_ref[...]
        tproj_ref[...] = (jnp.dot(_silu(tt), Wt_ref[...],
                                  preferred_element_type=jnp.float32)
                          + bt_ref[...])

    xb = x_ref[...]
    a1[...] += jnp.sum(xb, axis=0, keepdims=True)
    a2[...] += jnp.sum(xb * xb, axis=0, keepdims=True)

    @pl.when(i == GRID - 1)
    def _fin():
        sums_ref[...] = jnp.concatenate(
            [a1[...], a2[...], jnp.zeros((6, C), jnp.float32)], axis=0)


def _p2_body(x_ref, b_ref, sums1_ref, g1_ref, be1_ref, W1_ref, b1_ref,
             tproj_ref, sums2_ref, mu1_s, r1_s, a1, a2):
    i = pl.program_id(0)

    @pl.when(i == 0)
    def _init():
        s = sums1_ref[...]
        mu = s[0:1, :] / N
        var = s[1:2, :] / N - mu * mu
        mu1_s[...] = mu
        r1_s[...] = jax.lax.rsqrt(var + 1e-5)
        a1[...] = jnp.zeros_like(a1)
        a2[...] = jnp.zeros_like(a2)

    h = _h_tile(x_ref[...], b_ref[0, 0, :], mu1_s[...], r1_s[...],
                g1_ref[...], be1_ref[...], W1_ref[...], b1_ref[...],
                tproj_ref[...])
    a1[...] += jnp.sum(h, axis=0, keepdims=True)
    a2[...] += jnp.sum(h * h, axis=0, keepdims=True)

    @pl.when(i == GRID - 1)
    def _fin():
        sums2_ref[...] = jnp.concatenate(
            [a1[...], a2[...], jnp.zeros((6, C), jnp.float32)], axis=0)


def _p3_body(x_ref, b_ref, sums1_ref, sums2_ref, g1_ref, be1_ref, W1_ref,
             b1_ref, tproj_ref, g2_ref, be2_ref, W2_ref, b2_ref, Wd_ref,
             bd_ref, y_ref, mu1_s, r1_s, mu2_s, r2_s):
    i = pl.program_id(0)

    @pl.when(i == 0)
    def _init():
        s1 = sums1_ref[...]
        mu = s1[0:1, :] / N
        var = s1[1:2, :] / N - mu * mu
        mu1_s[...] = mu
        r1_s[...] = jax.lax.rsqrt(var + 1e-5)
        s2 = sums2_ref[...]
        mu2 = s2[0:1, :] / N
        var2 = s2[1:2, :] / N - mu2 * mu2
        mu2_s[...] = mu2
        r2_s[...] = jax.lax.rsqrt(var2 + 1e-5)

    xb = x_ref[...]
    h = _h_tile(xb, b_ref[0, 0, :], mu1_s[...], r1_s[...], g1_ref[...],
                be1_ref[...], W1_ref[...], b1_ref[...], tproj_ref[...])
    a = (h - mu2_s[...]) * r2_s[...] * g2_ref[...] + be2_ref[...]
    a = _silu(a)
    h2 = jnp.dot(a, W2_ref[...], preferred_element_type=jnp.float32) + b2_ref[...]
    hres = h2 + xb
    y_ref[...] = (jnp.dot(hres, Wd_ref[...], preferred_element_type=jnp.float32)
                  + bd_ref[...])


def _sc_scatter_body(y_hbm, ids_hbm, zeros_hbm, out_hbm, idx_v, data_v, acc_sh):
    cid = lax.axis_index("c")
    sid = lax.axis_index("s")
    wid = sid * 2 + cid

    pltpu.sync_copy(zeros_hbm.at[pl.ds(sid * RSUB, RSUB)],
                    acc_sh.at[pl.ds(sid * RSUB, RSUB)])
    plsc.subcore_barrier()

    pltpu.sync_copy(ids_hbm.at[wid], idx_v)
    base = wid * CPW * CHUNK

    def body(c, carry):
        pltpu.sync_copy(y_hbm.at[pl.ds(base + c * CHUNK, CHUNK)], data_v)
        pltpu.sync_copy(data_v, acc_sh.at[idx_v.at[c]], add=True)
        return carry

    lax.fori_loop(0, CPW, body, 0)
    plsc.subcore_barrier()

    pltpu.sync_copy(acc_sh.at[pl.ds(sid * RSUB, RSUB)],
                    out_hbm.at[pl.ds(cid * RPAD + sid * RSUB, RSUB)])


def _p4_body(p_ref, out_ref):
    out_ref[...] = p_ref[0] + p_ref[1]


def kernel(x, t, b, pool_ids, g1, be1, W1, b1, Wt, bt, g2, be2, W2, b2, Wd, bd):
    f32 = jnp.float32
    b_i = b.astype(jnp.int32).reshape(GRID, 1, TILE)
    ids = pool_ids.astype(jnp.int32)
    ids_pad = jnp.concatenate(
        [ids, jnp.full((NP_PAD - N,), NPOOL, jnp.int32)]).reshape(NW, CPW, CHUNK)
    g1r, be1r, b1r = g1.reshape(1, C), be1.reshape(1, C), b1.reshape(1, C)
    g2r, be2r, b2r = g2.reshape(1, C), be2.reshape(1, C), b2.reshape(1, C)
    bdr = bd.reshape(1, C)
    btr = bt.reshape(1, 2 * C)

    xspec = pl.BlockSpec((TILE, C), lambda i: (i, 0))
    bspec = pl.BlockSpec((1, 1, TILE), lambda i: (i, 0, 0))
    full = lambda shp: pl.BlockSpec(shp, lambda i: tuple(0 for _ in shp))

    sums1, tproj = pl.pallas_call(
        _p1_body,
        grid=(GRID,),
        in_specs=[xspec, full((NB, C)), full((C, 2 * C)), full((1, 2 * C))],
        out_specs=[full((8, C)), full((NB, 2 * C))],
        out_shape=[jax.ShapeDtypeStruct((8, C), f32),
                   jax.ShapeDtypeStruct((NB, 2 * C), f32)],
        scratch_shapes=[pltpu.VMEM((1, C), f32), pltpu.VMEM((1, C), f32)],
    )(x, t, Wt, btr)

    sums2 = pl.pallas_call(
        _p2_body,
        grid=(GRID,),
        in_specs=[xspec, bspec, full((8, C)), full((1, C)), full((1, C)),
                  full((C, C)), full((1, C)), full((NB, 2 * C))],
        out_specs=full((8, C)),
        out_shape=jax.ShapeDtypeStruct((8, C), f32),
        scratch_shapes=[pltpu.VMEM((1, C), f32)] * 4,
    )(x, b_i, sums1, g1r, be1r, W1, b1r, tproj)

    y = pl.pallas_call(
        _p3_body,
        grid=(GRID,),
        in_specs=[xspec, bspec, full((8, C)), full((8, C)), full((1, C)),
                  full((1, C)), full((C, C)), full((1, C)), full((NB, 2 * C)),
                  full((1, C)), full((1, C)), full((C, C)), full((1, C)),
                  full((C, C)), full((1, C))],
        out_specs=xspec,
        out_shape=jax.ShapeDtypeStruct((NP_PAD, C), f32),
        scratch_shapes=[pltpu.VMEM((1, C), f32)] * 4,
    )(x, b_i, sums1, sums2, g1r, be1r, W1, b1r, tproj, g2r, be2r, W2, b2r,
      Wd, bdr)

    zeros_hbm = jnp.zeros((RPAD, C), f32)
    mesh = plsc.VectorSubcoreMesh(core_axis_name="c", subcore_axis_name="s")
    partials = pl.kernel(
        _sc_scatter_body,
        mesh=mesh,
        out_type=jax.ShapeDtypeStruct((2 * RPAD, C), f32),
        scratch_types=[
            pltpu.VMEM((CPW, CHUNK), jnp.int32),
            pltpu.VMEM((CHUNK, C), f32),
            pltpu.VMEM_SHARED((RPAD, C), f32),
        ],
    )(y, ids_pad, zeros_hbm)

    res = pl.pallas_call(
        _p4_body,
        grid=(16,),
        in_specs=[pl.BlockSpec((2, RSUB, C), lambda i: (0, i, 0))],
        out_specs=pl.BlockSpec((RSUB, C), lambda i: (i, 0)),
        out_shape=jax.ShapeDtypeStruct((RPAD, C), f32),
    )(partials.reshape(2, RPAD, C))

    return res[:NPOOL]


# speedup vs baseline: 3.5111x; 1.0177x over previous
import functools

import jax
import jax.numpy as jnp
from jax import lax
from jax.experimental import pallas as pl
from jax.experimental.pallas import tpu as pltpu
from jax.experimental.pallas import tpu_sc as plsc

N = 100000
C = 128
NB = 16
NPOOL = 12500

TILE = 2000
GRID = N // TILE

NW = 32
CHUNK = 64
CPW = 50
NP_PAD = NW * CPW * CHUNK
RPAD = 12800
RSUB = RPAD // 16

_bf = jnp.bfloat16


def _silu(v):
    return v * jax.nn.sigmoid(v)


def _h_tile(xb, bidx, A1, B1, W1b, b1, tproj):
    a = xb * A1 + B1
    a = _silu(a)
    h1 = jnp.dot(a.astype(_bf), W1b, preferred_element_type=jnp.float32) + b1
    onehot = (bidx[:, None] == lax.broadcasted_iota(jnp.int32, (TILE, NB), 1)
              ).astype(_bf)
    film = jnp.dot(onehot, tproj.astype(_bf), preferred_element_type=jnp.float32)
    return (1.0 + film[:, :C]) * h1 + film[:, C:]


def _p1_body(x_ref, t_ref, Wt_ref, bt_ref, ones_ref, sums_ref, tproj_ref, acc):
    i = pl.program_id(0)

    @pl.when(i == 0)
    def _init():
        acc[...] = jnp.zeros_like(acc)
        tt = t_ref[...]
        tproj_ref[...] = (jnp.dot(_silu(tt).astype(_bf),
                                  Wt_ref[...].astype(_bf),
                                  preferred_element_type=jnp.float32)
                          + bt_ref[...])

    xb = x_ref[...]
    z = jnp.concatenate([xb, xb * xb], axis=1).astype(_bf)
    acc[...] += jnp.dot(ones_ref[...], z, preferred_element_type=jnp.float32)

    @pl.when(i == GRID - 1)
    def _fin():
        sums_ref[...] = acc[...]


def _p2_body(x_ref, b_ref, sums1_ref, g1_ref, be1_ref, W1_ref, b1_ref,
             tproj_ref, ones_ref, sums2_ref, A1_s, B1_s, acc):
    i = pl.program_id(0)

    @pl.when(i == 0)
    def _init():
        s = sums1_ref[...]
        mu = s[0:1, :C] / N
        var = s[0:1, C:] / N - mu * mu
        r = jax.lax.rsqrt(var + 1e-5)
        A1_s[...] = r * g1_ref[...]
        B1_s[...] = be1_ref[...] - mu * r * g1_ref[...]
        acc[...] = jnp.zeros_like(acc)

    h = _h_tile(x_ref[...], b_ref[0, 0, :], A1_s[...], B1_s[...],
                W1_ref[...].astype(_bf), b1_ref[...], tproj_ref[...])
    z = jnp.concatenate([h, h * h], axis=1).astype(_bf)
    acc[...] += jnp.dot(ones_ref[...], z, preferred_element_type=jnp.float32)

    @pl.when(i == GRID - 1)
    def _fin():
        sums2_ref[...] = acc[...]


def _p3_body(x_ref, b_ref, sums1_ref, sums2_ref, g1_ref, be1_ref, W1_ref,
             b1_ref, tproj_ref, g2_ref, be2_ref, W2_ref, b2_ref, Wd_ref,
             bd_ref, y_ref, A1_s, B1_s, A2_s, B2_s):
    i = pl.program_id(0)

    @pl.when(i == 0)
    def _init():
        s1 = sums1_ref[...]
        mu = s1[0:1, :C] / N
        var = s1[0:1, C:] / N - mu * mu
        r = jax.lax.rsqrt(var + 1e-5)
        A1_s[...] = r * g1_ref[...]
        B1_s[...] = be1_ref[...] - mu * r * g1_ref[...]
        s2 = sums2_ref[...]
        mu2 = s2[0:1, :C] / N
        var2 = s2[0:1, C:] / N - mu2 * mu2
        r2 = jax.lax.rsqrt(var2 + 1e-5)
        A2_s[...] = r2 * g2_ref[...]
        B2_s[...] = be2_ref[...] - mu2 * r2 * g2_ref[...]

    xb = x_ref[...]
    h = _h_tile(xb, b_ref[0, 0, :], A1_s[...], B1_s[...],
                W1_ref[...].astype(_bf), b1_ref[...], tproj_ref[...])
    a = h * A2_s[...] + B2_s[...]
    a = _silu(a)
    h2 = jnp.dot(a.astype(_bf), W2_ref[...].astype(_bf),
                 preferred_element_type=jnp.float32) + b2_ref[...]
    hres = h2 + xb
    y_ref[...] = (jnp.dot(hres.astype(_bf), Wd_ref[...].astype(_bf),
                          preferred_element_type=jnp.float32) + bd_ref[...])


def _sc_scatter_body(y_hbm, ids_hbm, zeros_hbm, out_hbm, idx_v, d0, d1,
                     acc_sh, sem0, sem1):
    cid = lax.axis_index("c")
    sid = lax.axis_index("s")
    wid = sid * 2 + cid

    pltpu.sync_copy(zeros_hbm.at[pl.ds(sid * RSUB, RSUB)],
                    acc_sh.at[pl.ds(sid * RSUB, RSUB)])
    plsc.subcore_barrier()

    pltpu.sync_copy(ids_hbm.at[wid], idx_v)
    base = wid * CPW * CHUNK

    def chunk(c):
        return y_hbm.at[pl.ds(base + c * CHUNK, CHUNK)]

    pltpu.async_copy(chunk(0), d0, sem0)

    def body(i, carry):
        c0 = 2 * i
        pltpu.async_copy(chunk(c0 + 1), d1, sem1)
        pltpu.make_async_copy(chunk(c0), d0, sem0).wait()
        pltpu.sync_copy(d0, acc_sh.at[idx_v.at[c0]], add=True)
        pltpu.async_copy(chunk(c0 + 2), d0, sem0)
        pltpu.make_async_copy(chunk(c0 + 1), d1, sem1).wait()
        pltpu.sync_copy(d1, acc_sh.at[idx_v.at[c0 + 1]], add=True)
        return carry

    lax.fori_loop(0, CPW // 2 - 1, body, 0)
    pltpu.make_async_copy(chunk(CPW - 2), d0, sem0).wait()
    pltpu.sync_copy(d0, acc_sh.at[idx_v.at[CPW - 2]], add=True)
    pltpu.sync_copy(chunk(CPW - 1), d1)
    pltpu.sync_copy(d1, acc_sh.at[idx_v.at[CPW - 1]], add=True)

    plsc.subcore_barrier()
    pltpu.sync_copy(acc_sh.at[pl.ds(sid * RSUB, RSUB)],
                    out_hbm.at[pl.ds(cid * RPAD + sid * RSUB, RSUB)])


def _p4_body(p_ref, out_ref):
    out_ref[...] = p_ref[0] + p_ref[1]


def kernel(x, t, b, pool_ids, g1, be1, W1, b1, Wt, bt, g2, be2, W2, b2, Wd, bd):
    f32 = jnp.float32
    b_i = b.astype(jnp.int32).reshape(GRID, 1, TILE)
    ids = pool_ids.astype(jnp.int32)
    ids_pad = jnp.concatenate(
        [ids, jnp.full((NP_PAD - N,), NPOOL, jnp.int32)]).reshape(NW, CPW, CHUNK)
    g1r, be1r, b1r = g1.reshape(1, C), be1.reshape(1, C), b1.reshape(1, C)
    g2r, be2r, b2r = g2.reshape(1, C), be2.reshape(1, C), b2.reshape(1, C)
    bdr = bd.reshape(1, C)
    btr = bt.reshape(1, 2 * C)
    ones8 = jnp.ones((8, TILE), _bf)

    xspec = pl.BlockSpec((TILE, C), lambda i: (i, 0))
    bspec = pl.BlockSpec((1, 1, TILE), lambda i: (i, 0, 0))
    full = lambda shp: pl.BlockSpec(shp, lambda i: tuple(0 for _ in shp))

    sums1, tproj = pl.pallas_call(
        _p1_body,
        grid=(GRID,),
        in_specs=[xspec, full((NB, C)), full((C, 2 * C)), full((1, 2 * C)),
                  full((8, TILE))],
        out_specs=[full((8, 2 * C)), full((NB, 2 * C))],
        out_shape=[jax.ShapeDtypeStruct((8, 2 * C), f32),
                   jax.ShapeDtypeStruct((NB, 2 * C), f32)],
        scratch_shapes=[pltpu.VMEM((8, 2 * C), f32)],
    )(x, t, Wt, btr, ones8)

    sums2 = pl.pallas_call(
        _p2_body,
        grid=(GRID,),
        in_specs=[xspec, bspec, full((8, 2 * C)), full((1, C)), full((1, C)),
                  full((C, C)), full((1, C)), full((NB, 2 * C)),
                  full((8, TILE))],
        out_specs=full((8, 2 * C)),
        out_shape=jax.ShapeDtypeStruct((8, 2 * C), f32),
        scratch_shapes=[pltpu.VMEM((1, C), f32), pltpu.VMEM((1, C), f32),
                        pltpu.VMEM((8, 2 * C), f32)],
    )(x, b_i, sums1, g1r, be1r, W1, b1r, tproj, ones8)

    y = pl.pallas_call(
        _p3_body,
        grid=(GRID,),
        in_specs=[xspec, bspec, full((8, 2 * C)), full((8, 2 * C)),
                  full((1, C)), full((1, C)), full((C, C)), full((1, C)),
                  full((NB, 2 * C)), full((1, C)), full((1, C)), full((C, C)),
                  full((1, C)), full((C, C)), full((1, C))],
        out_specs=xspec,
        out_shape=jax.ShapeDtypeStruct((NP_PAD, C), f32),
        scratch_shapes=[pltpu.VMEM((1, C), f32)] * 4,
    )(x, b_i, sums1, sums2, g1r, be1r, W1, b1r, tproj, g2r, be2r, W2, b2r,
      Wd, bdr)

    zeros_hbm = jnp.zeros((RPAD, C), f32)
    mesh = plsc.VectorSubcoreMesh(core_axis_name="c", subcore_axis_name="s")
    partials = pl.kernel(
        _sc_scatter_body,
        mesh=mesh,
        out_type=jax.ShapeDtypeStruct((2 * RPAD, C), f32),
        scratch_types=[
            pltpu.VMEM((CPW, CHUNK), jnp.int32),
            pltpu.VMEM((CHUNK, C), f32),
            pltpu.VMEM((CHUNK, C), f32),
            pltpu.VMEM_SHARED((RPAD, C), f32),
            pltpu.SemaphoreType.DMA,
            pltpu.SemaphoreType.DMA,
        ],
    )(y, ids_pad, zeros_hbm)

    res = pl.pallas_call(
        _p4_body,
        grid=(16,),
        in_specs=[pl.BlockSpec((2, RSUB, C), lambda i: (0, i, 0))],
        out_specs=pl.BlockSpec((RSUB, C), lambda i: (i, 0)),
        out_shape=jax.ShapeDtypeStruct((RPAD, C), f32),
    )(partials.reshape(2, RPAD, C))

    return res[:NPOOL]


# speedup vs baseline: 4.2998x; 1.2246x over previous
import functools

import jax
import jax.numpy as jnp
from jax import lax
from jax.experimental import pallas as pl
from jax.experimental.pallas import tpu as pltpu
from jax.experimental.pallas import tpu_sc as plsc

N = 100000
C = 128
NB = 16
NPOOL = 12500

TILE = 2000
GRID = N // TILE

NW = 32
CHUNK = 64
CPW = 50
NP_PAD = NW * CPW * CHUNK
RPAD = 12800
RSUB = RPAD // 16

_bf = jnp.bfloat16


def _silu(v):
    return v * jax.nn.sigmoid(v)


def _fused_body(x_ref, b_ref, t_ref, Wt_ref, bt_ref, ones_ref, g1_ref,
                be1_ref, W1_ref, b1_ref, g2_ref, be2_ref, W2_ref, b2_ref,
                Wd_ref, bd_ref, y_ref, x_cache, h_cache, acc1, acc2,
                tproj_s, A1_s, B1_s, A2_s, B2_s):
    p = pl.program_id(0)
    i = pl.program_id(1)
    rows = pl.ds(i * TILE, TILE)

    @pl.when(jnp.logical_and(p == 0, i == 0))
    def _init0():
        acc1[...] = jnp.zeros_like(acc1)
        acc2[...] = jnp.zeros_like(acc2)
        tt = t_ref[...]
        tproj_s[...] = (jnp.dot(_silu(tt).astype(_bf),
                                Wt_ref[...].astype(_bf),
                                preferred_element_type=jnp.float32)
                        + bt_ref[...])

    @pl.when(p == 0)
    def _phase0():
        xb = x_ref[...]
        z = jnp.concatenate([xb, xb * xb], axis=1).astype(_bf)
        acc1[...] += jnp.dot(ones_ref[...], z,
                             preferred_element_type=jnp.float32)
        x_cache[rows, :] = xb.astype(_bf)

    @pl.when(jnp.logical_and(p == 1, i == 0))
    def _init1():
        s = acc1[...]
        mu = s[0:1, :C] / N
        var = s[0:1, C:] / N - mu * mu
        r = jax.lax.rsqrt(var + 1e-5)
        A1_s[...] = r * g1_ref[...]
        B1_s[...] = be1_ref[...] - mu * r * g1_ref[...]

    @pl.when(p == 1)
    def _phase1():
        xb = x_cache[rows, :].astype(jnp.float32)
        a = xb * A1_s[...] + B1_s[...]
        a = _silu(a)
        h1 = jnp.dot(a.astype(_bf), W1_ref[...].astype(_bf),
                     preferred_element_type=jnp.float32) + b1_ref[...]
        bidx = b_ref[0, 0, :]
        onehot = (bidx[:, None]
                  == lax.broadcasted_iota(jnp.int32, (TILE, NB), 1)).astype(_bf)
        film = jnp.dot(onehot, tproj_s[...].astype(_bf),
                       preferred_element_type=jnp.float32)
        h = (1.0 + film[:, :C]) * h1 + film[:, C:]
        z = jnp.concatenate([h, h * h], axis=1).astype(_bf)
        acc2[...] += jnp.dot(ones_ref[...], z,
                             preferred_element_type=jnp.float32)
        h_cache[rows, :] = h.astype(_bf)

    @pl.when(jnp.logical_and(p == 2, i == 0))
    def _init2():
        s = acc2[...]
        mu = s[0:1, :C] / N
        var = s[0:1, C:] / N - mu * mu
        r = jax.lax.rsqrt(var + 1e-5)
        A2_s[...] = r * g2_ref[...]
        B2_s[...] = be2_ref[...] - mu * r * g2_ref[...]

    @pl.when(p == 2)
    def _phase2():
        h = h_cache[rows, :].astype(jnp.float32)
        a = h * A2_s[...] + B2_s[...]
        a = _silu(a)
        h2 = jnp.dot(a.astype(_bf), W2_ref[...].astype(_bf),
                     preferred_element_type=jnp.float32) + b2_ref[...]
        hres = h2 + x_cache[rows, :].astype(jnp.float32)
        y = (jnp.dot(hres.astype(_bf), Wd_ref[...].astype(_bf),
                     preferred_element_type=jnp.float32) + bd_ref[...])
        y_ref[...] = y


def _sc_scatter_body(y_hbm, ids_hbm, zeros_hbm, out_hbm, idx_v, d0, d1,
                     acc_sh, sem0, sem1):
    cid = lax.axis_index("c")
    sid = lax.axis_index("s")
    wid = sid * 2 + cid

    pltpu.sync_copy(zeros_hbm.at[pl.ds(sid * RSUB, RSUB)],
                    acc_sh.at[pl.ds(sid * RSUB, RSUB)])
    plsc.subcore_barrier()

    pltpu.sync_copy(ids_hbm.at[wid], idx_v)
    base = wid * CPW * CHUNK

    def chunk(c):
        return y_hbm.at[pl.ds(base + c * CHUNK, CHUNK)]

    pltpu.async_copy(chunk(0), d0, sem0)

    def body(i, carry):
        c0 = 2 * i
        pltpu.async_copy(chunk(c0 + 1), d1, sem1)
        pltpu.make_async_copy(chunk(c0), d0, sem0).wait()
        pltpu.sync_copy(d0, acc_sh.at[idx_v.at[c0]], add=True)
        pltpu.async_copy(chunk(c0 + 2), d0, sem0)
        pltpu.make_async_copy(chunk(c0 + 1), d1, sem1).wait()
        pltpu.sync_copy(d1, acc_sh.at[idx_v.at[c0 + 1]], add=True)
        return carry

    lax.fori_loop(0, CPW // 2 - 1, body, 0)
    pltpu.make_async_copy(chunk(CPW - 2), d0, sem0).wait()
    pltpu.sync_copy(d0, acc_sh.at[idx_v.at[CPW - 2]], add=True)
    pltpu.sync_copy(chunk(CPW - 1), d1)
    pltpu.sync_copy(d1, acc_sh.at[idx_v.at[CPW - 1]], add=True)

    plsc.subcore_barrier()
    pltpu.sync_copy(acc_sh.at[pl.ds(sid * RSUB, RSUB)],
                    out_hbm.at[pl.ds(cid * RPAD + sid * RSUB, RSUB)])


def _combine_body(p_ref, out_ref):
    a = p_ref[0, :NPOOL, :].astype(jnp.float32)
    b = p_ref[1, :NPOOL, :].astype(jnp.float32)
    out_ref[...] = a + b


def kernel(x, t, b, pool_ids, g1, be1, W1, b1, Wt, bt, g2, be2, W2, b2, Wd, bd):
    f32 = jnp.float32
    b_i = b.astype(jnp.int32).reshape(GRID, 1, TILE)
    ids = pool_ids.astype(jnp.int32)
    ids_pad = jnp.concatenate(
        [ids, jnp.full((NP_PAD - N,), NPOOL, jnp.int32)]).reshape(NW, CPW, CHUNK)
    g1r, be1r, b1r = g1.reshape(1, C), be1.reshape(1, C), b1.reshape(1, C)
    g2r, be2r, b2r = g2.reshape(1, C), be2.reshape(1, C), b2.reshape(1, C)
    bdr = bd.reshape(1, C)
    btr = bt.reshape(1, 2 * C)
    ones8 = jnp.ones((8, TILE), _bf)

    xspec = pl.BlockSpec((TILE, C), lambda p, i: (jnp.where(p == 0, i, 0), 0))
    bspec = pl.BlockSpec((1, 1, TILE), lambda p, i: (i, 0, 0))
    full = lambda shp: pl.BlockSpec(shp, lambda p, i: tuple(0 for _ in shp))

    y = pl.pallas_call(
        _fused_body,
        grid=(3, GRID),
        in_specs=[xspec, bspec, full((NB, C)), full((C, 2 * C)),
                  full((1, 2 * C)), full((8, TILE)), full((1, C)),
                  full((1, C)), full((C, C)), full((1, C)), full((1, C)),
                  full((1, C)), full((C, C)), full((1, C)), full((C, C)),
                  full((1, C))],
        out_specs=pl.BlockSpec((TILE, C),
                               lambda p, i: (jnp.where(p == 2, i, 0), 0)),
        out_shape=jax.ShapeDtypeStruct((NP_PAD, C), f32),
        scratch_shapes=[
            pltpu.VMEM((N, C), _bf),
            pltpu.VMEM((N, C), _bf),
            pltpu.VMEM((8, 2 * C), f32),
            pltpu.VMEM((8, 2 * C), f32),
            pltpu.VMEM((NB, 2 * C), f32),
            pltpu.VMEM((1, C), f32), pltpu.VMEM((1, C), f32),
            pltpu.VMEM((1, C), f32), pltpu.VMEM((1, C), f32),
        ],
    )(x, b_i, t, Wt, btr, ones8, g1r, be1r, W1, b1r, g2r, be2r, W2, b2r,
      Wd, bdr)

    zeros_hbm = jnp.zeros((RPAD, C), f32)
    mesh = plsc.VectorSubcoreMesh(core_axis_name="c", subcore_axis_name="s")
    partials = pl.kernel(
        _sc_scatter_body,
        mesh=mesh,
        out_type=jax.ShapeDtypeStruct((2 * RPAD, C), f32),
        scratch_types=[
            pltpu.VMEM((CPW, CHUNK), jnp.int32),
            pltpu.VMEM((CHUNK, C), f32),
            pltpu.VMEM((CHUNK, C), f32),
            pltpu.VMEM_SHARED((RPAD, C), f32),
            pltpu.SemaphoreType.DMA,
            pltpu.SemaphoreType.DMA,
        ],
    )(y, ids_pad, zeros_hbm)

    out = pl.pallas_call(
        _combine_body,
        grid=(1,),
        in_specs=[pl.BlockSpec((2, RPAD, C), lambda i: (0, 0, 0))],
        out_specs=pl.BlockSpec((NPOOL, C), lambda i: (0, 0)),
        out_shape=jax.ShapeDtypeStruct((NPOOL, C), f32),
    )(partials.reshape(2, RPAD, C))

    return out
